# Initial kernel scaffold; baseline (speedup 1.0000x reference)
import jax
import jax.numpy as jnp
from jax.experimental import pallas as pl


def kernel(h, edge_f, dt, edge_w, dst_idx, time_freq, tp_W1, tp_b1, tp_W2, tp_b2, tp_W3, tp_b3, cf_W1, cf_b1, cf_W2, cf_b2, cf_W3, cf_b3, ln_g, ln_b, ln2_g, ln2_b):
    raise NotImplementedError("write your pallas kernel here")


# speedup vs baseline: 2.0516x; 2.0516x over previous
import jax
import jax.numpy as jnp
from jax.experimental import pallas as pl
from jax.experimental.pallas import tpu as pltpu

_B = 1000
_S = 128
_D = 128


def _edge_kernel(dt_ref, ew_ref, dstv_ref, dsts_ref, h_ref, ef_ref,
                 tfreq_ref, w1h_ref, w1e_ref, w1t_ref, b1_ref,
                 w2_ref, b2_ref, w3_ref, b3_ref,
                 h2_ref, deg_ref):
    i = pl.program_id(0)

    @pl.when(i == 0)
    def _init():
        h2_ref[...] = jnp.zeros_like(h2_ref)
        deg_ref[...] = jnp.zeros_like(deg_ref)

    f32 = jnp.float32
    tf = jnp.cos(dt_ref[...] * tfreq_ref[...])
    x = (jnp.dot(h_ref[...], w1h_ref[...], preferred_element_type=f32)
         + jnp.dot(ef_ref[...], w1e_ref[...], preferred_element_type=f32)
         + jnp.dot(tf, w1t_ref[...], preferred_element_type=f32)
         + b1_ref[...])
    x = jnp.maximum(x, 0.0)
    x = jnp.maximum(
        jnp.dot(x, w2_ref[...], preferred_element_type=f32) + b2_ref[...], 0.0)
    v = (jnp.dot(x, w3_ref[...], preferred_element_type=f32)
         + b3_ref[...]) * ew_ref[...]

    dst = dstv_ref[0]
    lo = dsts_ref[0, 0, 0]
    hi = dsts_ref[0, 0, _B - 1]
    base0 = (lo // 8) * 8

    def body(base):
        rel = dst - base
        m = (jax.lax.broadcasted_iota(jnp.int32, (_S, _B), 0)
             == rel).astype(f32)
        h2_ref[pl.ds(base, _S), :] += jnp.dot(m, v, preferred_element_type=f32)
        deg_ref[pl.ds(base, _S), :] += jnp.broadcast_to(
            jnp.sum(m, axis=1, keepdims=True), (_S, _D))
        return base + _S

    jax.lax.while_loop(lambda b: b <= hi, body, base0)


def _ln(x, g, b):
    mu = jnp.mean(x, axis=1, keepdims=True)
    xc = x - mu
    var = jnp.mean(xc * xc, axis=1, keepdims=True)
    return xc * jax.lax.rsqrt(var + 1e-5) * g + b


def _combine_kernel(h2_ref, deg_ref, hd_ref, w1a_ref, w1b_ref, b1_ref,
                    w2_ref, b2_ref, w3_ref, b3_ref,
                    lng_ref, lnb_ref, ln2g_ref, ln2b_ref, o_ref):
    f32 = jnp.float32
    h2 = h2_ref[...]
    h1 = h2 / jnp.maximum(deg_ref[...], 1.0)
    x = (jnp.dot(h1, w1a_ref[...], preferred_element_type=f32)
         + jnp.dot(hd_ref[...], w1b_ref[...], preferred_element_type=f32)
         + b1_ref[...])
    x = jnp.maximum(x, 0.0)
    x = jnp.maximum(
        jnp.dot(x, w2_ref[...], preferred_element_type=f32) + b2_ref[...], 0.0)
    rst = jnp.dot(x, w3_ref[...], preferred_element_type=f32) + b3_ref[...]
    o_ref[...] = (_ln(rst, lng_ref[...], lnb_ref[...])
                  + _ln(h2, ln2g_ref[...], ln2b_ref[...]))


def kernel(h, edge_f, dt, edge_w, dst_idx, time_freq,
           tp_W1, tp_b1, tp_W2, tp_b2, tp_W3, tp_b3,
           cf_W1, cf_b1, cf_W2, cf_b2, cf_W3, cf_b3,
           ln_g, ln_b, ln2_g, ln2_b):
    E = edge_f.shape[0]
    n_dst = h.shape[0] - E
    grid_e = E // _B
    n_pad = ((n_dst + _S + 7) // _S + 1) * _S

    dt2 = dt.reshape(E, 1)
    dst3 = dst_idx.astype(jnp.int32).reshape(grid_e, 1, _B)
    d_node = h.shape[1]
    d_edge = edge_f.shape[1]
    d_time = time_freq.shape[0]
    tpad = _D - d_time
    tfreq_p = jnp.concatenate([time_freq, jnp.zeros((tpad,), jnp.float32)]
                              ).reshape(1, _D)
    w1h = tp_W1[:d_node]
    w1e = tp_W1[d_node:d_node + d_edge]
    w1t = jnp.concatenate(
        [tp_W1[d_node + d_edge:], jnp.zeros((tpad, _D), jnp.float32)], axis=0)
    row = lambda a: a.reshape(1, -1)

    const = lambda shape: pl.BlockSpec(shape, lambda i: (0, 0))
    h2, deg = pl.pallas_call(
        _edge_kernel,
        grid=(grid_e,),
        in_specs=[
            pl.BlockSpec((_B, 1), lambda i: (i, 0)),
            pl.BlockSpec((_B, 1), lambda i: (i, 0)),
            pl.BlockSpec((1, 1, _B), lambda i: (i, 0, 0)),
            pl.BlockSpec((1, 1, _B), lambda i: (i, 0, 0),
                         memory_space=pltpu.SMEM),
            pl.BlockSpec((_B, _D), lambda i: (n_dst // _B + i, 0)),
            pl.BlockSpec((_B, d_edge), lambda i: (i, 0)),
            const((1, _D)),
            const((_D, _D)),
            const((d_edge, _D)),
            const((_D, _D)),
            const((1, _D)),
            const((_D, _D)),
            const((1, _D)),
            const((_D, _D)),
            const((1, _D)),
        ],
        out_specs=[
            pl.BlockSpec((n_pad, _D), lambda i: (0, 0)),
            pl.BlockSpec((n_pad, _D), lambda i: (0, 0)),
        ],
        out_shape=[jax.ShapeDtypeStruct((n_pad, _D), jnp.float32)] * 2,
    )(dt2, edge_w, dst3, dst3, h, edge_f, tfreq_p,
      w1h, w1e, w1t, row(tp_b1), tp_W2, row(tp_b2), tp_W3, row(tp_b3))

    grid_n = n_dst // _B
    blk = lambda: pl.BlockSpec((_B, _D), lambda i: (i, 0))
    out = pl.pallas_call(
        _combine_kernel,
        grid=(grid_n,),
        in_specs=[
            blk(), blk(), blk(),
            const((_D, _D)), const((_D, _D)), const((1, _D)),
            const((_D, _D)), const((1, _D)),
            const((_D, _D)), const((1, _D)),
            const((1, _D)), const((1, _D)), const((1, _D)), const((1, _D)),
        ],
        out_specs=blk(),
        out_shape=jax.ShapeDtypeStruct((n_dst, _D), jnp.float32),
    )(h2, deg, h, cf_W1[:_D], cf_W1[_D:], row(cf_b1),
      cf_W2, row(cf_b2), cf_W3, row(cf_b3),
      row(ln_g), row(ln_b), row(ln2_g), row(ln2_b))
    return out
